# baseline (speedup 1.0000x reference)
import jax
import jax.numpy as jnp
from jax import lax
from jax.experimental import pallas as pl
from jax.experimental.pallas import tpu as pltpu

_MXU_ROWS = 256
_CHUNK = 8
_UNROLL = 2


def _batched_lstm_head_kernel(xt_ref, wih_ref, whh_ref, b_ref, wlin_ref,
                              blin_ref, out_ref, xpre_ref):
    seq_len, batch, input_size = xt_ref.shape
    Hp = whh_ref.shape[0]
    wdtype = whh_ref.dtype
    n_grp = max(1, batch // _MXU_ROWS)
    rows = batch // n_grp
    chunk = _CHUNK if seq_len % _CHUNK == 0 else 1
    n_chunks = seq_len // chunk

    def chunk_body(ci, carry):
        xc = xt_ref[pl.ds(ci * chunk, chunk)].reshape(chunk * batch,
                                                      input_size)
        xpre_ref[...] = (jnp.dot(xc, wih_ref[...],
                                 preferred_element_type=jnp.float32)
                         + b_ref[...])

        for k in range(chunk):
            new = []
            for j in range(n_grp):
                h, c = carry[2 * j], carry[2 * j + 1]
                base = k * batch + j * rows
                pre = (xpre_ref[base:base + rows, :]
                       + jnp.dot(h.astype(wdtype), whh_ref[...],
                                 preferred_element_type=jnp.float32))

                sig = jax.nn.sigmoid(pre[:, :3 * Hp])
                i_g = sig[:, 0:Hp]
                f_g = sig[:, Hp:2 * Hp]
                o_g = sig[:, 2 * Hp:3 * Hp]
                g_g = jnp.tanh(pre[:, 3 * Hp:])

                c_new = f_g * c + i_g * g_g
                h_new = o_g * jnp.tanh(c_new)
                new += [h_new, c_new]
            carry = tuple(new)
        return carry

    init = tuple(jnp.zeros((rows, Hp), jnp.float32) for _ in range(2 * n_grp))
    carry = lax.fori_loop(0, n_chunks, chunk_body, init, unroll=_UNROLL)

    for j in range(n_grp):
        h_last = carry[2 * j]
        out_ref[j * rows:(j + 1) * rows, :] = (
            jnp.dot(h_last.astype(wlin_ref.dtype), wlin_ref[...],
                    preferred_element_type=jnp.float32) + blin_ref[...])


def _full_spec(arr):
    ndim = arr.ndim
    return pl.BlockSpec(arr.shape, lambda n: (0,) * ndim)


def _pick_batch(n_seq):
    for b in (512, 256, 128, 64, 32, 16, 8):
        if n_seq % b == 0:
            return b
    return n_seq


@jax.jit
def kernel(xs, wih_f, whh_f, bias_f, wlin_f, blin_f):
    n_seq, seq_len, input_size = xs.shape
    Hp = whh_f.shape[0]
    output_size = wlin_f.shape[1]
    B = _pick_batch(n_seq)
    chunk = _CHUNK if seq_len % _CHUNK == 0 else 1

    xt = jnp.transpose(xs.astype(whh_f.dtype), (1, 0, 2))

    return pl.pallas_call(
        _batched_lstm_head_kernel,
        out_shape=jax.ShapeDtypeStruct((n_seq, output_size), jnp.float32),
        grid=(n_seq // B,),
        in_specs=[
            pl.BlockSpec((seq_len, B, input_size), lambda n: (0, n, 0)),
            _full_spec(wih_f),
            _full_spec(whh_f),
            _full_spec(bias_f),
            _full_spec(wlin_f),
            _full_spec(blin_f),
        ],
        out_specs=pl.BlockSpec((B, output_size), lambda n: (n, 0)),
        scratch_shapes=[pltpu.VMEM((chunk * B, 4 * Hp), jnp.float32)],
        compiler_params=pltpu.CompilerParams(
            dimension_semantics=("parallel",)),
    )(xt, wih_f, whh_f, bias_f, wlin_f, blin_f)


# speedup vs baseline: 1.3632x; 1.3632x over previous
import jax
import jax.numpy as jnp
from jax import lax
from jax.experimental import pallas as pl
from jax.experimental.pallas import tpu as pltpu

_MXU_ROWS = 256
_UNROLL = 32


def _batched_lstm_head_kernel(xt_ref, wih_ref, whh_ref, b_ref, wlin_ref,
                              blin_ref, out_ref):
    seq_len, batch, _ = xt_ref.shape
    Hp = whh_ref.shape[0]
    wdtype = whh_ref.dtype
    n_grp = max(1, batch // _MXU_ROWS)
    rows = batch // n_grp

    def step(t, carry):
        xt = xt_ref[t]
        new = []
        for j in range(n_grp):
            h, c = carry[2 * j], carry[2 * j + 1]
            pre = (jnp.dot(xt[j * rows:(j + 1) * rows], wih_ref[...],
                           preferred_element_type=jnp.float32)
                   + jnp.dot(h.astype(wdtype), whh_ref[...],
                             preferred_element_type=jnp.float32)
                   + b_ref[...])

            sig = jnp.tanh(pre[:, :3 * Hp] * 0.5) * 0.5 + 0.5
            i_g = sig[:, 0:Hp]
            f_g = sig[:, Hp:2 * Hp]
            o_g = sig[:, 2 * Hp:3 * Hp]
            g_g = jnp.tanh(pre[:, 3 * Hp:])

            c_new = f_g * c + i_g * g_g
            h_new = o_g * jnp.tanh(c_new)
            new += [h_new, c_new]
        return tuple(new)

    init = tuple(jnp.zeros((rows, Hp), jnp.float32) for _ in range(2 * n_grp))
    carry = lax.fori_loop(0, seq_len, step, init, unroll=_UNROLL)

    for j in range(n_grp):
        h_last = carry[2 * j]
        out_ref[j * rows:(j + 1) * rows, :] = (
            jnp.dot(h_last.astype(wlin_ref.dtype), wlin_ref[...],
                    preferred_element_type=jnp.float32) + blin_ref[...])


def _full_spec(arr):
    ndim = arr.ndim
    return pl.BlockSpec(arr.shape, lambda n: (0,) * ndim)


def _pick_batch(n_seq):
    for b in (512, 256, 128, 64, 32, 16, 8):
        if n_seq % b == 0:
            return b
    return n_seq


@jax.jit
def kernel(xs, wih_f, whh_f, bias_f, wlin_f, blin_f):
    n_seq, seq_len, input_size = xs.shape
    output_size = wlin_f.shape[1]
    B = _pick_batch(n_seq)

    xt = jnp.transpose(xs.astype(whh_f.dtype), (1, 0, 2))

    return pl.pallas_call(
        _batched_lstm_head_kernel,
        out_shape=jax.ShapeDtypeStruct((n_seq, output_size), jnp.float32),
        grid=(n_seq // B,),
        in_specs=[
            pl.BlockSpec((seq_len, B, input_size), lambda n: (0, n, 0)),
            _full_spec(wih_f),
            _full_spec(whh_f),
            _full_spec(bias_f),
            _full_spec(wlin_f),
            _full_spec(blin_f),
        ],
        out_specs=pl.BlockSpec((B, output_size), lambda n: (n, 0)),
        compiler_params=pltpu.CompilerParams(
            dimension_semantics=("parallel",)),
    )(xt, wih_f, whh_f, bias_f, wlin_f, blin_f)
